# Initial kernel scaffold; baseline (speedup 1.0000x reference)
import jax
import jax.numpy as jnp
from jax.experimental import pallas as pl


def kernel(detail_pos, weights):
    raise NotImplementedError("write your pallas kernel here")


# speedup vs baseline: 9.1572x; 9.1572x over previous
import functools

import jax
import jax.numpy as jnp
from jax import lax
from jax.experimental import pallas as pl
from jax.experimental.pallas import tpu as pltpu
from jax.experimental.pallas import tpu_sc as plsc

D = 128
NC = 2
NS = 16
NW = NC * NS
CHUNK = 128


@functools.partial(jax.jit, static_argnames=("b_per_w",))
def _sc_gather(flat_idx, weights, b_per_w):
    B = flat_idx.shape[0]
    n_chunks = b_per_w // CHUNK
    mesh = plsc.VectorSubcoreMesh(core_axis_name="c", subcore_axis_name="s")

    @functools.partial(
        pl.kernel,
        out_type=jax.ShapeDtypeStruct((B, D), jnp.float32),
        mesh=mesh,
        scratch_types=[
            pltpu.VMEM((2, CHUNK), jnp.int32),
            pltpu.VMEM((2, CHUNK, D), jnp.float32),
            pltpu.SemaphoreType.DMA((2,)),
        ],
    )
    def k(idx_hbm, table_hbm, out_hbm, idx_v, rows_v, sems):
        wid = lax.axis_index("s") * NC + lax.axis_index("c")
        base = wid * b_per_w

        def start(g, slot):
            off = base + g * CHUNK
            pltpu.sync_copy(idx_hbm.at[pl.ds(off, CHUNK)], idx_v.at[slot])
            pltpu.async_copy(
                table_hbm.at[idx_v.at[slot]], rows_v.at[slot], sems.at[slot]
            )

        start(0, 0)

        def finish(g, slot):
            pltpu.make_async_copy(
                table_hbm.at[idx_v.at[slot]], rows_v.at[slot], sems.at[slot]
            ).wait()
            off = base + g * CHUNK
            pltpu.sync_copy(rows_v.at[slot], out_hbm.at[pl.ds(off, CHUNK)])

        def body(g2, _):
            for s in (0, 1):
                g = g2 * 2 + s

                @pl.when(g + 1 < n_chunks)
                def _():
                    start(g + 1, 1 - s)

                finish(g, s)
            return 0

        lax.fori_loop(0, n_chunks // 2, body, 0)

    return k(flat_idx, weights)


def kernel(detail_pos, weights):
    shape = detail_pos.shape
    flat = detail_pos.reshape(-1).astype(jnp.int32)
    out = _sc_gather(flat, weights.astype(jnp.float32), flat.shape[0] // NW)
    return out.reshape(shape + (weights.shape[-1],))


# speedup vs baseline: 9.9533x; 1.0869x over previous
import functools

import jax
import jax.numpy as jnp
from jax import lax
from jax.experimental import pallas as pl
from jax.experimental.pallas import tpu as pltpu
from jax.experimental.pallas import tpu_sc as plsc

D = 128
NC = 2
NS = 16
NW = NC * NS
CHUNK = 128
NBUF = 4


@functools.partial(jax.jit, static_argnames=("n_chunks",))
def _sc_gather(idx3, weights, n_chunks):
    B = NW * n_chunks * CHUNK
    mesh = plsc.VectorSubcoreMesh(core_axis_name="c", subcore_axis_name="s")

    @functools.partial(
        pl.kernel,
        out_type=jax.ShapeDtypeStruct((B, D), jnp.float32),
        mesh=mesh,
        scratch_types=[
            pltpu.VMEM((n_chunks, CHUNK), jnp.int32),
            pltpu.VMEM((NBUF, CHUNK, D), jnp.float32),
            pltpu.SemaphoreType.DMA((NBUF,)),
            pltpu.SemaphoreType.DMA((NBUF,)),
        ],
    )
    def k(idx_hbm, table_hbm, out_hbm, idx_v, rows_v, gsem, ssem):
        wid = lax.axis_index("s") * NC + lax.axis_index("c")
        base = wid * (n_chunks * CHUNK)
        pltpu.sync_copy(idx_hbm.at[wid], idx_v)

        def gather(g, s):
            pltpu.async_copy(
                table_hbm.at[idx_v.at[g]], rows_v.at[s], gsem.at[s]
            )

        def wait_gather(g, s):
            pltpu.make_async_copy(
                table_hbm.at[idx_v.at[g]], rows_v.at[s], gsem.at[s]
            ).wait()

        def store(g, s):
            off = base + g * CHUNK
            pltpu.async_copy(
                rows_v.at[s], out_hbm.at[pl.ds(off, CHUNK)], ssem.at[s]
            )

        def wait_store(g, s):
            off = base + g * CHUNK
            pltpu.make_async_copy(
                rows_v.at[s], out_hbm.at[pl.ds(off, CHUNK)], ssem.at[s]
            ).wait()

        for s in range(NBUF):
            gather(s, s)

        def body(t, _):
            for s in range(NBUF):
                g = t * NBUF + s
                wait_gather(g, s)
                store(g, s)

                @pl.when(g + NBUF < n_chunks)
                def _():
                    wait_store(g, s)
                    gather(g + NBUF, s)

            return 0

        lax.fori_loop(0, n_chunks // NBUF, body, 0)

        for s in range(NBUF):
            wait_store(n_chunks - NBUF + s, s)

    return k(idx3, weights)


def kernel(detail_pos, weights):
    shape = detail_pos.shape
    flat = detail_pos.reshape(-1).astype(jnp.int32)
    n_chunks = flat.shape[0] // (NW * CHUNK)
    idx3 = flat.reshape(NW, n_chunks, CHUNK)
    out = _sc_gather(idx3, weights.astype(jnp.float32), n_chunks)
    return out.reshape(shape + (weights.shape[-1],))


# speedup vs baseline: 17.3622x; 1.7444x over previous
import functools

import jax
import jax.numpy as jnp
from jax import lax
from jax.experimental import pallas as pl
from jax.experimental.pallas import tpu as pltpu
from jax.experimental.pallas import tpu_sc as plsc

D = 128
NC = 2
NS = 16
NW = NC * NS
CHUNK = 128
NBUF = 2


@functools.partial(jax.jit, static_argnames=("n_chunks",))
def _sc_gather(idx3, weights, n_chunks):
    B = NW * n_chunks * CHUNK
    mesh = plsc.VectorSubcoreMesh(core_axis_name="c", subcore_axis_name="s")

    @functools.partial(
        pl.kernel,
        out_type=jax.ShapeDtypeStruct((B, D), jnp.float32),
        mesh=mesh,
        scratch_types=[
            pltpu.VMEM((n_chunks, CHUNK), jnp.int32),
            pltpu.VMEM((NBUF, CHUNK, D), jnp.float32),
            pltpu.VMEM_SHARED((8192, D), jnp.float32),
            pltpu.SemaphoreType.DMA((NBUF,)),
            pltpu.SemaphoreType.DMA((NBUF,)),
        ],
    )
    def k(idx_hbm, table_hbm, out_hbm, idx_v, rows_v, table_sh, gsem, ssem):
        wid = lax.axis_index("s") * NC + lax.axis_index("c")
        base = wid * (n_chunks * CHUNK)

        sid = lax.axis_index("s")
        rows_per_tile = 8192 // NS
        pltpu.sync_copy(
            table_hbm.at[pl.ds(sid * rows_per_tile, rows_per_tile)],
            table_sh.at[pl.ds(sid * rows_per_tile, rows_per_tile)],
        )
        pltpu.sync_copy(idx_hbm.at[wid], idx_v)
        plsc.subcore_barrier()

        def gather(g, s):
            pltpu.async_copy(
                table_sh.at[idx_v.at[g]], rows_v.at[s], gsem.at[s]
            )

        def wait_gather(g, s):
            pltpu.make_async_copy(
                table_sh.at[idx_v.at[g]], rows_v.at[s], gsem.at[s]
            ).wait()

        def store(g, s):
            off = base + g * CHUNK
            pltpu.async_copy(
                rows_v.at[s], out_hbm.at[pl.ds(off, CHUNK)], ssem.at[s]
            )

        def wait_store(g, s):
            off = base + g * CHUNK
            pltpu.make_async_copy(
                rows_v.at[s], out_hbm.at[pl.ds(off, CHUNK)], ssem.at[s]
            ).wait()

        for s in range(NBUF):
            gather(s, s)

        def body(t, _):
            for s in range(NBUF):
                g = t * NBUF + s
                wait_gather(g, s)
                store(g, s)

                @pl.when(g + NBUF < n_chunks)
                def _():
                    wait_store(g, s)
                    gather(g + NBUF, s)

            return 0

        lax.fori_loop(0, n_chunks // NBUF, body, 0)

        for s in range(NBUF):
            wait_store(n_chunks - NBUF + s, s)

    return k(idx3, weights)


def kernel(detail_pos, weights):
    shape = detail_pos.shape
    flat = detail_pos.reshape(-1).astype(jnp.int32)
    n_chunks = flat.shape[0] // (NW * CHUNK)
    idx3 = flat.reshape(NW, n_chunks, CHUNK)
    out = _sc_gather(idx3, weights.astype(jnp.float32), n_chunks)
    return out.reshape(shape + (weights.shape[-1],))


# speedup vs baseline: 18.0304x; 1.0385x over previous
import functools

import jax
import jax.numpy as jnp
from jax import lax
from jax.experimental import pallas as pl
from jax.experimental.pallas import tpu as pltpu
from jax.experimental.pallas import tpu_sc as plsc

D = 128
NC = 2
NS = 16
NW = NC * NS
CHUNK = 80
NBUF = 4
IB = 20


@functools.partial(jax.jit, static_argnames=("n_blocks",))
def _sc_gather(idx4, weights, n_blocks):
    B = NW * n_blocks * IB * CHUNK
    total = n_blocks * IB
    mesh = plsc.VectorSubcoreMesh(core_axis_name="c", subcore_axis_name="s")

    @functools.partial(
        pl.kernel,
        out_type=jax.ShapeDtypeStruct((B, D), jnp.float32),
        mesh=mesh,
        scratch_types=[
            pltpu.VMEM((2, IB, CHUNK), jnp.int32),
            pltpu.VMEM((NBUF, CHUNK, D), jnp.float32),
            pltpu.VMEM_SHARED((8192, D), jnp.float32),
            pltpu.SemaphoreType.DMA((NBUF,)),
            pltpu.SemaphoreType.DMA((NBUF,)),
            pltpu.SemaphoreType.DMA((2,)),
        ],
    )
    def k(idx_hbm, table_hbm, out_hbm, idx_v, rows_v, table_sh, gsem, ssem, isem):
        wid = lax.axis_index("s") * NC + lax.axis_index("c")
        base = wid * (total * CHUNK)

        sid = lax.axis_index("s")
        rows_per_tile = 8192 // NS
        pltpu.sync_copy(
            table_hbm.at[pl.ds(sid * rows_per_tile, rows_per_tile)],
            table_sh.at[pl.ds(sid * rows_per_tile, rows_per_tile)],
        )
        pltpu.sync_copy(idx_hbm.at[wid, 0], idx_v.at[0])
        plsc.subcore_barrier()

        def load_idx(blk, p):
            return pltpu.make_async_copy(
                idx_hbm.at[wid, blk], idx_v.at[p], isem.at[p]
            )

        def gather(p, j, s):
            return pltpu.make_async_copy(
                table_sh.at[idx_v.at[p].at[j]], rows_v.at[s], gsem.at[s]
            )

        def store(g, s):
            off = base + g * CHUNK
            return pltpu.make_async_copy(
                rows_v.at[s], out_hbm.at[pl.ds(off, CHUNK)], ssem.at[s]
            )

        for j in range(NBUF):
            gather(0, j, j).start()

        def body(k_, _):
            b = lax.rem(k_, 2)

            @pl.when(k_ + 1 < n_blocks)
            def _():
                load_idx(k_ + 1, 1 - b).start()

            for j in range(IB):
                g = k_ * IB + j
                s = j % NBUF
                gather(b, j, s).wait()
                store(g, s).start()

                if j == IB - NBUF:
                    @pl.when(k_ + 1 < n_blocks)
                    def _():
                        load_idx(k_ + 1, 1 - b).wait()

                @pl.when(g + NBUF < total)
                def _():
                    store(g, s).wait()
                    if j < IB - NBUF:
                        gather(b, j + NBUF, s).start()
                    else:
                        gather(1 - b, j + NBUF - IB, s).start()

            return 0

        lax.fori_loop(0, n_blocks, body, 0)

        for s in range(NBUF):
            store(total - NBUF + s, s).wait()

    return k(idx4, weights)


def kernel(detail_pos, weights):
    shape = detail_pos.shape
    flat = detail_pos.reshape(-1).astype(jnp.int32)
    n_blocks = flat.shape[0] // (NW * IB * CHUNK)
    idx4 = flat.reshape(NW, n_blocks, IB, CHUNK)
    out = _sc_gather(idx4, weights.astype(jnp.float32), n_blocks)
    return out.reshape(shape + (weights.shape[-1],))
